# baseline (speedup 1.0000x reference)
import functools

import jax
import jax.numpy as jnp
from jax import lax
from jax.experimental import pallas as pl
from jax.experimental.pallas import tpu as pltpu
from jax.experimental.pallas import tpu_sc as plsc

N = 10000
E = 320000
D = 128
C = 40
L = 6
EPS = 1e-5

NC = 2
NS = 16
NW = NC * NS
CHUNK = 96
NCHUNK = 108
HC = NCHUNK // 2
EPW = NCHUNK * CHUNK
EPAD = NW * EPW
NA = N + 8
NTC = 10
CPT = N // NTC

_mesh = plsc.VectorSubcoreMesh(core_axis_name="c", subcore_axis_name="s")


def _fill_vmem(ref, rows, cols, vec16):
    def row(i, _):
        for j in range(cols // 16):
            ref[i, pl.ds(j * 16, 16)] = vec16
        return 0

    lax.fori_loop(0, rows, row, 0)


def _zero_acc(sid, buf0, acc_sh):
    nfull, rem = divmod(CPT, CHUNK)

    @pl.when(sid < NTC)
    def _():
        for b in range(nfull):
            pltpu.sync_copy(buf0, acc_sh.at[pl.ds(sid * CPT + b * CHUNK, CHUNK)])
        if rem:
            pltpu.sync_copy(
                buf0.at[pl.ds(0, rem)],
                acc_sh.at[pl.ds(sid * CPT + nfull * CHUNK, rem)],
            )


def _copy_out(sid, cid, acc_sh, out_hbm):
    @pl.when(sid < NTC)
    def _():
        pltpu.sync_copy(
            acc_sh.at[pl.ds(sid * CPT, CPT)],
            out_hbm.at[cid, pl.ds(sid * CPT, CPT)],
        )


@functools.partial(
    pl.kernel,
    out_type=jax.ShapeDtypeStruct((NC, N, D), jnp.float32),
    mesh=_mesh,
    scratch_types=[
        pltpu.VMEM((HC * CHUNK,), jnp.int32),
        pltpu.VMEM((HC, CHUNK), jnp.int32),
        pltpu.VMEM((3, CHUNK, D), jnp.float32),
        pltpu.VMEM_SHARED((NA, D), jnp.float32),
        pltpu.SemaphoreType.DMA,
        pltpu.SemaphoreType.DMA,
        pltpu.SemaphoreType.DMA,
        pltpu.SemaphoreType.DMA,
    ],
)
def _sc_agg(h_hbm, src_hbm, dst_hbm, out_hbm, src_v, dst_v, buf_v,
            acc_sh, semg, sems0, sems1, sems2):
    cid = lax.axis_index("c")
    sid = lax.axis_index("s")
    wid = sid * NC + cid

    bufs = [buf_v.at[0], buf_v.at[1], buf_v.at[2]]
    sems = [sems0, sems1, sems2]
    _fill_vmem(bufs[0], CHUNK, D, jnp.zeros((16,), jnp.float32))
    _zero_acc(sid, bufs[0], acc_sh)
    plsc.subcore_barrier()

    for half in range(2):
        pltpu.sync_copy(
            src_hbm.at[pl.ds(wid * EPW + half * (HC * CHUNK), HC * CHUNK)],
            src_v,
        )
        pltpu.sync_copy(dst_hbm.at[wid, half], dst_v)

        pltpu.async_copy(h_hbm.at[src_v.at[pl.ds(0, CHUNK)]], bufs[0], semg)

        def triple(k, _):
            for b in range(3):
                j = 3 * k + b
                pltpu.make_async_copy(
                    h_hbm.at[src_v.at[pl.ds(pl.multiple_of(j * CHUNK, 8), CHUNK)]],
                    bufs[b], semg,
                ).wait()

                @pl.when(j >= 2)
                def _():
                    pltpu.make_async_copy(
                        bufs[b - 2], acc_sh.at[dst_v.at[j]], sems[b - 2]
                    ).wait()

                @pl.when(j + 1 < HC)
                def _():
                    pltpu.async_copy(
                        h_hbm.at[
                            src_v.at[pl.ds(pl.multiple_of((j + 1) * CHUNK, 8), CHUNK)]
                        ],
                        bufs[(b + 1) % 3], semg,
                    )

                pltpu.async_copy(bufs[b], acc_sh.at[dst_v.at[j]], sems[b], add=True)
            return 0

        lax.fori_loop(0, HC // 3, triple, 0)
        for b in (1, 2):
            pltpu.make_async_copy(
                bufs[b], acc_sh.at[dst_v.at[HC - 1]], sems[b]
            ).wait()

    plsc.subcore_barrier()
    _copy_out(sid, cid, acc_sh, out_hbm)


@functools.partial(
    pl.kernel,
    out_type=jax.ShapeDtypeStruct((NC, N, D), jnp.float32),
    mesh=_mesh,
    scratch_types=[
        pltpu.VMEM((HC, CHUNK), jnp.int32),
        pltpu.VMEM((CHUNK, D), jnp.float32),
        pltpu.VMEM_SHARED((NA, D), jnp.float32),
    ],
)
def _sc_ones(dst_hbm, out_hbm, dst_v, buf_v, acc_sh):
    cid = lax.axis_index("c")
    sid = lax.axis_index("s")
    wid = sid * NC + cid

    _fill_vmem(buf_v, CHUNK, D, jnp.zeros((16,), jnp.float32))
    _zero_acc(sid, buf_v, acc_sh)
    plsc.subcore_barrier()

    _fill_vmem(buf_v, CHUNK, D, jnp.ones((16,), jnp.float32))
    for half in range(2):
        pltpu.sync_copy(dst_hbm.at[wid, half], dst_v)

        def body(j, _):
            pltpu.sync_copy(buf_v, acc_sh.at[dst_v.at[j]], add=True)
            return 0

        lax.fori_loop(0, HC, body, 0)

    plsc.subcore_barrier()
    _copy_out(sid, cid, acc_sh, out_hbm)


BR = 1000


def _dense_body(p_ref, invb_ref, h_ref, wl_ref, wr_ref, b_ref, o_ref):
    agg = (p_ref[0] + p_ref[1]) * invb_ref[...]
    z = jnp.dot(agg, wl_ref[...], precision=lax.Precision.HIGHEST,
                preferred_element_type=jnp.float32)
    z = z + jnp.dot(h_ref[...], wr_ref[...], precision=lax.Precision.HIGHEST,
                    preferred_element_type=jnp.float32)
    z = z + b_ref[...]
    o_ref[...] = jnp.maximum(z, 0.0)


_dense = pl.pallas_call(
    _dense_body,
    grid=(N // BR,),
    in_specs=[
        pl.BlockSpec((NC, BR, D), lambda i: (0, i, 0)),
        pl.BlockSpec((BR, D), lambda i: (i, 0)),
        pl.BlockSpec((BR, D), lambda i: (i, 0)),
        pl.BlockSpec((D, D), lambda i: (0, 0)),
        pl.BlockSpec((D, D), lambda i: (0, 0)),
        pl.BlockSpec((1, D), lambda i: (0, 0)),
    ],
    out_specs=pl.BlockSpec((BR, D), lambda i: (i, 0)),
    out_shape=jax.ShapeDtypeStruct((N, D), jnp.float32),
)


def _pool_body(h_ref, wlin_ref, blin_ref, o_ref, acc_ref):
    i = pl.program_id(0)

    @pl.when(i == 0)
    def _():
        acc_ref[...] = jnp.zeros_like(acc_ref)

    acc_ref[0:1, :] += jnp.sum(h_ref[...], axis=0, keepdims=True)

    @pl.when(i == pl.num_programs(0) - 1)
    def _():
        pooled = acc_ref[0:1, :] * (1.0 / N)
        o_ref[...] = (
            jnp.dot(pooled, wlin_ref[...], precision=lax.Precision.HIGHEST,
                    preferred_element_type=jnp.float32)
            + blin_ref[...]
        )


_pool = pl.pallas_call(
    _pool_body,
    grid=(N // BR,),
    in_specs=[
        pl.BlockSpec((BR, D), lambda i: (i, 0)),
        pl.BlockSpec((D, C), lambda i: (0, 0)),
        pl.BlockSpec((1, C), lambda i: (0, 0)),
    ],
    out_specs=pl.BlockSpec((1, C), lambda i: (0, 0)),
    out_shape=jax.ShapeDtypeStruct((1, C), jnp.float32),
    scratch_shapes=[pltpu.VMEM((8, D), jnp.float32)],
)


@jax.jit
def kernel(x, edge_index, Wl, bl, Wr, bn_g, bn_b, bn_rm, bn_rv, Wlin, blin):
    pad = EPAD - E
    src = jnp.concatenate([edge_index[0], jnp.zeros((pad,), jnp.int32)])
    dst = jnp.concatenate(
        [edge_index[1], jnp.full((pad,), N, jnp.int32)]
    ).reshape(NW, 2, HC, CHUNK)

    s = bn_g * lax.rsqrt(bn_rv + EPS)
    wl_f = Wl * s[:, None, :]
    wr_f = Wr * s[:, None, :]
    b_f = (bl - bn_rm) * s + bn_b

    cntp = _sc_ones(dst)
    cnt = cntp[0, :, 0] + cntp[1, :, 0]
    inv = 1.0 / jnp.maximum(cnt, 1.0)
    invb = jnp.broadcast_to(inv[:, None], (N, D))

    h = x
    for i in range(L):
        p = _sc_agg(h, src, dst)
        h = _dense(p, invb, h, wl_f[i], wr_f[i], b_f[i][None, :])

    return _pool(h, Wlin, blin[None, :])


# speedup vs baseline: 1.3779x; 1.3779x over previous
import functools

import jax
import jax.numpy as jnp
from jax import lax
from jax.experimental import pallas as pl
from jax.experimental.pallas import tpu as pltpu
from jax.experimental.pallas import tpu_sc as plsc

N = 10000
E = 320000
D = 128
C = 40
L = 6
EPS = 1e-5

NC = 2
NS = 16
NW = NC * NS
CHUNK = 128
NCHUNK = 80
HC = NCHUNK // 2
EPW = NCHUNK * CHUNK
EPAD = NW * EPW
NA = N + 8
RCPAD = 0
NTC = 10
CPT = N // NTC

_mesh = plsc.VectorSubcoreMesh(core_axis_name="c", subcore_axis_name="s")


def _fill_vmem(ref, rows, cols, vec16):
    def row(i, _):
        for j in range(cols // 16):
            ref[i, pl.ds(j * 16, 16)] = vec16
        return 0

    lax.fori_loop(0, rows, row, 0)


def _zero_acc(sid, buf0, acc_sh):
    @pl.when(sid < NTC)
    def _():
        for b in range(7):
            pltpu.sync_copy(buf0, acc_sh.at[pl.ds(sid * CPT + b * CHUNK, CHUNK)])
        pltpu.sync_copy(
            buf0.at[pl.ds(0, CPT - 7 * CHUNK)],
            acc_sh.at[pl.ds(sid * CPT + 7 * CHUNK, CPT - 7 * CHUNK)],
        )


def _copy_out(sid, cid, acc_sh, out_hbm):
    @pl.when(sid < NTC)
    def _():
        pltpu.sync_copy(
            acc_sh.at[pl.ds(sid * CPT, CPT)],
            out_hbm.at[cid, pl.ds(sid * CPT, CPT)],
        )


@functools.partial(
    pl.kernel,
    out_type=jax.ShapeDtypeStruct((NC, N, D), jnp.float32),
    mesh=_mesh,
    scratch_types=[
        pltpu.VMEM((HC, CHUNK), jnp.int32),
        pltpu.VMEM((HC, CHUNK), jnp.int32),
        pltpu.VMEM((2, CHUNK, D), jnp.float32),
        pltpu.VMEM_SHARED((NA, D), jnp.float32),
        pltpu.SemaphoreType.DMA,
        pltpu.SemaphoreType.DMA,
    ],
)
def _sc_agg(h_hbm, src_hbm, dst_hbm, out_hbm, src_v, dst_v, buf_v, acc_sh, semg, sems):
    cid = lax.axis_index("c")
    sid = lax.axis_index("s")
    wid = sid * NC + cid

    bufs = [buf_v.at[0], buf_v.at[1]]
    _fill_vmem(bufs[0], CHUNK, D, jnp.zeros((16,), jnp.float32))
    _zero_acc(sid, bufs[0], acc_sh)
    plsc.subcore_barrier()

    for half in range(2):
        pltpu.sync_copy(src_hbm.at[wid, pl.ds(half * HC, HC)], src_v)
        pltpu.sync_copy(dst_hbm.at[wid, pl.ds(half * HC, HC)], dst_v)

        pltpu.async_copy(h_hbm.at[src_v.at[0]], bufs[0], semg)

        def pair(k, _):
            for b in range(2):
                j = 2 * k + b
                pltpu.make_async_copy(
                    h_hbm.at[src_v.at[j]], bufs[b], semg
                ).wait()

                @pl.when(j > 0)
                def _():
                    pltpu.make_async_copy(
                        bufs[1 - b], acc_sh.at[dst_v.at[j]], sems
                    ).wait()

                @pl.when(j + 1 < HC)
                def _():
                    pltpu.async_copy(h_hbm.at[src_v.at[j + 1]], bufs[1 - b], semg)

                pltpu.async_copy(bufs[b], acc_sh.at[dst_v.at[j]], sems, add=True)
            return 0

        lax.fori_loop(0, HC // 2, pair, 0)
        pltpu.make_async_copy(
            bufs[1], acc_sh.at[dst_v.at[HC - 1]], sems
        ).wait()

    plsc.subcore_barrier()
    _copy_out(sid, cid, acc_sh, out_hbm)


CW = 128


@functools.partial(
    pl.kernel,
    out_type=jax.ShapeDtypeStruct((NC, N, CW), jnp.float32),
    mesh=_mesh,
    scratch_types=[
        pltpu.VMEM((HC, CHUNK), jnp.int32),
        pltpu.VMEM((CHUNK, CW), jnp.float32),
        pltpu.VMEM_SHARED((NA, CW), jnp.float32),
    ],
)
def _sc_ones(dst_hbm, out_hbm, dst_v, buf_v, acc_sh):
    cid = lax.axis_index("c")
    sid = lax.axis_index("s")
    wid = sid * NC + cid

    _fill_vmem(buf_v, CHUNK, CW, jnp.zeros((16,), jnp.float32))

    @pl.when(sid < NTC)
    def _():
        for b in range(7):
            pltpu.sync_copy(buf_v, acc_sh.at[pl.ds(sid * CPT + b * CHUNK, CHUNK)])
        pltpu.sync_copy(
            buf_v.at[pl.ds(0, CPT - 7 * CHUNK)],
            acc_sh.at[pl.ds(sid * CPT + 7 * CHUNK, CPT - 7 * CHUNK)],
        )

    plsc.subcore_barrier()

    _fill_vmem(buf_v, CHUNK, CW, jnp.ones((16,), jnp.float32))
    for half in range(2):
        pltpu.sync_copy(dst_hbm.at[wid, pl.ds(half * HC, HC)], dst_v)

        def body(j, _):
            pltpu.sync_copy(buf_v, acc_sh.at[dst_v.at[j]], add=True)
            return 0

        lax.fori_loop(0, HC, body, 0)

    plsc.subcore_barrier()

    @pl.when(sid < NTC)
    def _():
        pltpu.sync_copy(
            acc_sh.at[pl.ds(sid * CPT, CPT)],
            out_hbm.at[cid, pl.ds(sid * CPT, CPT)],
        )


BR = 1000


def _dense_body(p_ref, invb_ref, h_ref, wl_ref, wr_ref, b_ref, o_ref):
    agg = (p_ref[0] + p_ref[1]) * invb_ref[...]
    z = jnp.dot(agg, wl_ref[...], precision=lax.Precision.HIGHEST,
                preferred_element_type=jnp.float32)
    z = z + jnp.dot(h_ref[...], wr_ref[...], precision=lax.Precision.HIGHEST,
                    preferred_element_type=jnp.float32)
    z = z + b_ref[...]
    o_ref[...] = jnp.maximum(z, 0.0)


_dense = pl.pallas_call(
    _dense_body,
    grid=(N // BR,),
    in_specs=[
        pl.BlockSpec((NC, BR, D), lambda i: (0, i, 0)),
        pl.BlockSpec((BR, D), lambda i: (i, 0)),
        pl.BlockSpec((BR, D), lambda i: (i, 0)),
        pl.BlockSpec((D, D), lambda i: (0, 0)),
        pl.BlockSpec((D, D), lambda i: (0, 0)),
        pl.BlockSpec((1, D), lambda i: (0, 0)),
    ],
    out_specs=pl.BlockSpec((BR, D), lambda i: (i, 0)),
    out_shape=jax.ShapeDtypeStruct((N, D), jnp.float32),
)


def _pool_body(h_ref, wlin_ref, blin_ref, o_ref, acc_ref):
    i = pl.program_id(0)

    @pl.when(i == 0)
    def _():
        acc_ref[...] = jnp.zeros_like(acc_ref)

    acc_ref[0:1, :] += jnp.sum(h_ref[...], axis=0, keepdims=True)

    @pl.when(i == pl.num_programs(0) - 1)
    def _():
        pooled = acc_ref[0:1, :] * (1.0 / N)
        o_ref[...] = (
            jnp.dot(pooled, wlin_ref[...], precision=lax.Precision.HIGHEST,
                    preferred_element_type=jnp.float32)
            + blin_ref[...]
        )


_pool = pl.pallas_call(
    _pool_body,
    grid=(N // BR,),
    in_specs=[
        pl.BlockSpec((BR, D), lambda i: (i, 0)),
        pl.BlockSpec((D, C), lambda i: (0, 0)),
        pl.BlockSpec((1, C), lambda i: (0, 0)),
    ],
    out_specs=pl.BlockSpec((1, C), lambda i: (0, 0)),
    out_shape=jax.ShapeDtypeStruct((1, C), jnp.float32),
    scratch_shapes=[pltpu.VMEM((8, D), jnp.float32)],
)


@jax.jit
def kernel(x, edge_index, Wl, bl, Wr, bn_g, bn_b, bn_rm, bn_rv, Wlin, blin):
    pad = EPAD - E
    src = jnp.concatenate(
        [edge_index[0], jnp.zeros((pad,), jnp.int32)]
    ).reshape(NW, NCHUNK, CHUNK)
    dst = jnp.concatenate(
        [edge_index[1], jnp.full((pad,), N, jnp.int32)]
    ).reshape(NW, NCHUNK, CHUNK)

    s = bn_g * lax.rsqrt(bn_rv + EPS)
    wl_f = Wl * s[:, None, :]
    wr_f = Wr * s[:, None, :]
    b_f = (bl - bn_rm) * s + bn_b

    cntp = _sc_ones(dst)
    cnt = cntp[0, :, 0] + cntp[1, :, 0]
    inv = 1.0 / jnp.maximum(cnt, 1.0)
    invb = jnp.broadcast_to(inv[:, None], (N, D))

    h = x
    for i in range(L):
        p = _sc_agg(h, src, dst)
        h = _dense(p, invb, h, wl_f[i], wr_f[i], b_f[i][None, :])

    return _pool(h, Wlin, blin[None, :])


# speedup vs baseline: 1.3786x; 1.0005x over previous
import functools

import jax
import jax.numpy as jnp
from jax import lax
from jax.experimental import pallas as pl
from jax.experimental.pallas import tpu as pltpu
from jax.experimental.pallas import tpu_sc as plsc

N = 10000
E = 320000
D = 128
C = 40
L = 6
EPS = 1e-5

NC = 2
NS = 16
NW = NC * NS
CHUNK = 128
NCHUNK = 80
HC = NCHUNK // 2
EPW = NCHUNK * CHUNK
EPAD = NW * EPW
NA = N + 8
NTC = 10
CPT = N // NTC

_mesh = plsc.VectorSubcoreMesh(core_axis_name="c", subcore_axis_name="s")


def _fill_vmem(ref, rows, cols, vec16):
    def row(i, _):
        for j in range(cols // 16):
            ref[i, pl.ds(j * 16, 16)] = vec16
        return 0

    lax.fori_loop(0, rows, row, 0)


def _zero_acc(sid, buf0, acc_sh):
    @pl.when(sid < NTC)
    def _():
        for b in range(7):
            pltpu.sync_copy(buf0, acc_sh.at[pl.ds(sid * CPT + b * CHUNK, CHUNK)])
        pltpu.sync_copy(
            buf0.at[pl.ds(0, CPT - 7 * CHUNK)],
            acc_sh.at[pl.ds(sid * CPT + 7 * CHUNK, CPT - 7 * CHUNK)],
        )


def _copy_out(sid, cid, acc_sh, out_hbm):
    @pl.when(sid < NTC)
    def _():
        pltpu.sync_copy(
            acc_sh.at[pl.ds(sid * CPT, CPT)],
            out_hbm.at[cid, pl.ds(sid * CPT, CPT)],
        )


@functools.partial(
    pl.kernel,
    out_type=jax.ShapeDtypeStruct((NC, N, D), jnp.float32),
    mesh=_mesh,
    scratch_types=[
        pltpu.VMEM((HC, CHUNK), jnp.int32),
        pltpu.VMEM((HC, CHUNK), jnp.int32),
        pltpu.VMEM((2, CHUNK, D), jnp.float32),
        pltpu.VMEM_SHARED((NA, D), jnp.float32),
        pltpu.SemaphoreType.DMA,
        pltpu.SemaphoreType.DMA,
    ],
)
def _sc_agg(h_hbm, src_hbm, dst_hbm, out_hbm, src_v, dst_v, buf_v, acc_sh, semg, sems):
    cid = lax.axis_index("c")
    sid = lax.axis_index("s")
    wid = sid * NC + cid

    bufs = [buf_v.at[0], buf_v.at[1]]
    _fill_vmem(bufs[0], CHUNK, D, jnp.zeros((16,), jnp.float32))
    _zero_acc(sid, bufs[0], acc_sh)
    plsc.subcore_barrier()

    for half in range(2):
        pltpu.sync_copy(src_hbm.at[wid, pl.ds(half * HC, HC)], src_v)
        pltpu.sync_copy(dst_hbm.at[wid, pl.ds(half * HC, HC)], dst_v)

        pltpu.async_copy(h_hbm.at[src_v.at[0]], bufs[0], semg)

        def pair(k, _):
            for b in range(2):
                j = 2 * k + b
                pltpu.make_async_copy(
                    h_hbm.at[src_v.at[j]], bufs[b], semg
                ).wait()

                @pl.when(j > 0)
                def _():
                    pltpu.make_async_copy(
                        bufs[1 - b], acc_sh.at[dst_v.at[j]], sems
                    ).wait()

                @pl.when(j + 1 < HC)
                def _():
                    pltpu.async_copy(h_hbm.at[src_v.at[j + 1]], bufs[1 - b], semg)

                pltpu.async_copy(bufs[b], acc_sh.at[dst_v.at[j]], sems, add=True)
            return 0

        lax.fori_loop(0, HC // 2, pair, 0)
        pltpu.make_async_copy(
            bufs[1], acc_sh.at[dst_v.at[HC - 1]], sems
        ).wait()

    plsc.subcore_barrier()
    _copy_out(sid, cid, acc_sh, out_hbm)


CW = 128


@functools.partial(
    pl.kernel,
    out_type=jax.ShapeDtypeStruct((NC, N, CW), jnp.float32),
    mesh=_mesh,
    scratch_types=[
        pltpu.VMEM((HC, CHUNK), jnp.int32),
        pltpu.VMEM((CHUNK, CW), jnp.float32),
        pltpu.VMEM_SHARED((NA, CW), jnp.float32),
    ],
)
def _sc_ones(dst_hbm, out_hbm, dst_v, buf_v, acc_sh):
    cid = lax.axis_index("c")
    sid = lax.axis_index("s")
    wid = sid * NC + cid

    _fill_vmem(buf_v, CHUNK, CW, jnp.zeros((16,), jnp.float32))

    @pl.when(sid < NTC)
    def _():
        for b in range(7):
            pltpu.sync_copy(buf_v, acc_sh.at[pl.ds(sid * CPT + b * CHUNK, CHUNK)])
        pltpu.sync_copy(
            buf_v.at[pl.ds(0, CPT - 7 * CHUNK)],
            acc_sh.at[pl.ds(sid * CPT + 7 * CHUNK, CPT - 7 * CHUNK)],
        )

    plsc.subcore_barrier()

    _fill_vmem(buf_v, CHUNK, CW, jnp.ones((16,), jnp.float32))
    for half in range(2):
        pltpu.sync_copy(dst_hbm.at[wid, pl.ds(half * HC, HC)], dst_v)

        def body(j, _):
            pltpu.sync_copy(buf_v, acc_sh.at[dst_v.at[j]], add=True)
            return 0

        lax.fori_loop(0, HC, body, 0)

    plsc.subcore_barrier()

    @pl.when(sid < NTC)
    def _():
        pltpu.sync_copy(
            acc_sh.at[pl.ds(sid * CPT, CPT)],
            out_hbm.at[cid, pl.ds(sid * CPT, CPT)],
        )


BR = 1000


def _dense_body(p_ref, invb_ref, h_ref, wl_ref, wr_ref, b_ref, o_ref):
    agg = (p_ref[0] + p_ref[1]) * invb_ref[...]
    z = jnp.dot(agg, wl_ref[...], precision=lax.Precision.HIGHEST,
                preferred_element_type=jnp.float32)
    z = z + jnp.dot(h_ref[...], wr_ref[...], precision=lax.Precision.HIGHEST,
                    preferred_element_type=jnp.float32)
    z = z + b_ref[...]
    o_ref[...] = jnp.maximum(z, 0.0)


_dense = pl.pallas_call(
    _dense_body,
    grid=(N // BR,),
    in_specs=[
        pl.BlockSpec((NC, BR, D), lambda i: (0, i, 0)),
        pl.BlockSpec((BR, D), lambda i: (i, 0)),
        pl.BlockSpec((BR, D), lambda i: (i, 0)),
        pl.BlockSpec((D, D), lambda i: (0, 0)),
        pl.BlockSpec((D, D), lambda i: (0, 0)),
        pl.BlockSpec((1, D), lambda i: (0, 0)),
    ],
    out_specs=pl.BlockSpec((BR, D), lambda i: (i, 0)),
    out_shape=jax.ShapeDtypeStruct((N, D), jnp.float32),
)


def _pool_body(h_ref, wlin_ref, blin_ref, o_ref, acc_ref):
    i = pl.program_id(0)

    @pl.when(i == 0)
    def _():
        acc_ref[...] = jnp.zeros_like(acc_ref)

    acc_ref[0:1, :] += jnp.sum(h_ref[...], axis=0, keepdims=True)

    @pl.when(i == pl.num_programs(0) - 1)
    def _():
        pooled = acc_ref[0:1, :] * (1.0 / N)
        o_ref[...] = (
            jnp.dot(pooled, wlin_ref[...], precision=lax.Precision.HIGHEST,
                    preferred_element_type=jnp.float32)
            + blin_ref[...]
        )


_pool = pl.pallas_call(
    _pool_body,
    grid=(N // BR,),
    in_specs=[
        pl.BlockSpec((BR, D), lambda i: (i, 0)),
        pl.BlockSpec((D, C), lambda i: (0, 0)),
        pl.BlockSpec((1, C), lambda i: (0, 0)),
    ],
    out_specs=pl.BlockSpec((1, C), lambda i: (0, 0)),
    out_shape=jax.ShapeDtypeStruct((1, C), jnp.float32),
    scratch_shapes=[pltpu.VMEM((8, D), jnp.float32)],
)


@jax.jit
def kernel(x, edge_index, Wl, bl, Wr, bn_g, bn_b, bn_rm, bn_rv, Wlin, blin):
    pad = EPAD - E
    src = jnp.concatenate(
        [edge_index[0], jnp.zeros((pad,), jnp.int32)]
    ).reshape(NW, NCHUNK, CHUNK)
    dst = jnp.concatenate(
        [edge_index[1], jnp.full((pad,), N, jnp.int32)]
    ).reshape(NW, NCHUNK, CHUNK)

    s = bn_g * lax.rsqrt(bn_rv + EPS)
    wl_f = Wl * s[:, None, :]
    wr_f = Wr * s[:, None, :]
    b_f = (bl - bn_rm) * s + bn_b

    cntp = _sc_ones(dst)
    cnt = cntp[0, :, 0] + cntp[1, :, 0]
    inv = 1.0 / jnp.maximum(cnt, 1.0)
    invb = jnp.broadcast_to(inv[:, None], (N, D))

    h = x
    for i in range(L):
        p = _sc_agg(h, src, dst)
        h = _dense(p, invb, h, wl_f[i], wr_f[i], b_f[i][None, :])

    return _pool(h, Wlin, blin[None, :])
